# baseline (speedup 1.0000x reference)
import functools

import jax
import jax.numpy as jnp
from jax import lax
from jax.experimental import pallas as pl
from jax.experimental.pallas import tpu as pltpu
from jax.experimental.pallas import tpu_sc as plsc

NUM_EMBEDDINGS = 1024
CODE_DIM = 256
NUM_CODEBOOKS = 4
COMMITMENT_COST = 0.25
EPS = 1e-10

ROWS_PER_TILE = 2048


def _vq_tc_body(z_ref, e_ref, idx_ref, commit_ref, ppl_ref, counts_scr,
                commit_scr):
    i = pl.program_id(0)
    z = z_ref[...]
    e = e_ref[...]
    e2 = jnp.sum(e * e, axis=1, keepdims=True).reshape(1, -1)
    z2 = jnp.sum(z * z, axis=1, keepdims=True)
    s_neg = lax.dot_general(z * -2.0, e, (((1,), (1,)), ((), ())),
                            preferred_element_type=jnp.float32)
    d = (z2 + s_neg) + e2
    R, K = d.shape
    md = jnp.min(d, axis=1, keepdims=True)
    iota_f = lax.broadcasted_iota(jnp.int32, (1, K), 1).astype(jnp.float32)
    idxf = jnp.min(jnp.where(d == md, iota_f, 2048.0), axis=1,
                   keepdims=True)
    idx_ref[...] = idxf.astype(jnp.int32)

    tile_commit = jnp.sum(md)
    onehot = (iota_f == idxf).astype(jnp.float32)
    tile_counts = lax.dot_general(jnp.ones((1, R), jnp.float32), onehot,
                                  (((1,), (0,)), ((), ())),
                                  preferred_element_type=jnp.float32)

    @pl.when(i == 0)
    def _init():
        counts_scr[...] = tile_counts
        commit_scr[0] = tile_commit

    @pl.when(i > 0)
    def _acc():
        counts_scr[...] += tile_counts
        commit_scr[0] += tile_commit

    @pl.when(i == pl.num_programs(0) - 1)
    def _fin():
        total_rows = R * pl.num_programs(0)
        p = counts_scr[...] / total_rows
        ent = -jnp.sum(p * jnp.log(p + EPS), axis=1, keepdims=True)
        ppl_ref[...] = jnp.exp(ent)
        commit_ref[...] = jnp.full((1, 1), commit_scr[0], jnp.float32)


def _vq_distance_argmin(flat_z, embed):
    n, dc = flat_z.shape
    k = embed.shape[0]
    g = n // ROWS_PER_TILE
    idx3, commit, ppl = pl.pallas_call(
        _vq_tc_body,
        grid=(g,),
        in_specs=[
            pl.BlockSpec((ROWS_PER_TILE, dc), lambda i: (i, 0)),
            pl.BlockSpec((k, dc), lambda i: (0, 0)),
        ],
        out_specs=[
            pl.BlockSpec((ROWS_PER_TILE, 1), lambda i: (i, 0)),
            pl.BlockSpec((1, 1), lambda i: (0, 0)),
            pl.BlockSpec((1, 1), lambda i: (0, 0)),
        ],
        out_shape=[
            jax.ShapeDtypeStruct((n, 1), jnp.int32),
            jax.ShapeDtypeStruct((1, 1), jnp.float32),
            jax.ShapeDtypeStruct((1, 1), jnp.float32),
        ],
        scratch_shapes=[
            pltpu.VMEM((1, k), jnp.float32),
            pltpu.SMEM((1,), jnp.float32),
        ],
        compiler_params=pltpu.CompilerParams(
            dimension_semantics=("arbitrary",)),
    )(flat_z, embed)
    return idx3.reshape(n), commit[0, 0], ppl[0, 0]


_SC_CHUNK = 128


def _make_sc_gather(n_rows, dc):
    info = plsc.get_sparse_core_info()
    nw = info.num_cores * info.num_subcores
    b_per_w = n_rows // nw
    n_ch = b_per_w // _SC_CHUNK
    mesh = plsc.VectorSubcoreMesh(core_axis_name="c", subcore_axis_name="s")

    @functools.partial(
        pl.kernel, mesh=mesh,
        out_type=jax.ShapeDtypeStruct((n_rows, dc), jnp.float32),
        scratch_types=[
            pltpu.VMEM((n_ch, _SC_CHUNK), jnp.int32),
            pltpu.VMEM((_SC_CHUNK, dc), jnp.float32),
            pltpu.VMEM((_SC_CHUNK, dc), jnp.float32),
            pltpu.SemaphoreType.DMA,
            pltpu.SemaphoreType.DMA,
        ],
    )
    def _gather(idx_hbm, table_hbm, out_hbm, idx_v, rows_a, rows_b, sem_a,
                sem_b):
        wid = lax.axis_index("s") * info.num_cores + lax.axis_index("c")
        base = wid * b_per_w
        pltpu.sync_copy(idx_hbm.at[pl.ds(wid * n_ch, n_ch)], idx_v)
        bufs = (rows_a, rows_b)
        sems = (sem_a, sem_b)
        copies = [None] * n_ch
        copies[0] = pltpu.async_copy(table_hbm.at[idx_v.at[0]], bufs[0],
                                     sems[0])
        for c in range(n_ch):
            if c + 1 < n_ch:
                copies[c + 1] = pltpu.async_copy(
                    table_hbm.at[idx_v.at[c + 1]], bufs[(c + 1) % 2],
                    sems[(c + 1) % 2])
            copies[c].wait()
            pltpu.sync_copy(bufs[c % 2],
                            out_hbm.at[pl.ds(base + c * _SC_CHUNK, _SC_CHUNK)])

    return _gather


def kernel(z_bt, embed):
    k, dc = embed.shape
    flat_z = z_bt.reshape(-1, dc)
    n = flat_z.shape[0]

    flat_indices, commit_sum, perplexity = _vq_distance_argmin(flat_z, embed)

    idx2d = flat_indices.reshape(-1, _SC_CHUNK)
    quantized_flat = _make_sc_gather(n, dc)(idx2d, embed)
    quantized_st = quantized_flat.reshape(z_bt.shape)

    commitment_loss = commit_sum / z_bt.size
    codebook_loss = jnp.zeros((), dtype=z_bt.dtype)
    loss = COMMITMENT_COST * commitment_loss
    indices = flat_indices.reshape(-1, NUM_CODEBOOKS)
    return (quantized_st, indices, loss, codebook_loss, commitment_loss,
            perplexity)


# speedup vs baseline: 1.1704x; 1.1704x over previous
import functools

import jax
import jax.numpy as jnp
from jax import lax
from jax.experimental import pallas as pl
from jax.experimental.pallas import tpu as pltpu
from jax.experimental.pallas import tpu_sc as plsc

NUM_EMBEDDINGS = 1024
CODE_DIM = 256
NUM_CODEBOOKS = 4
COMMITMENT_COST = 0.25
EPS = 1e-10

def _vq_tc_body(z_ref, e_ref, idx_ref, commit_ref, ppl_ref, counts_scr,
                commit_scr):
    i = pl.program_id(0)
    e = e_ref[...]
    K, Dc = e.shape
    Rb = z_ref.shape[0]
    e2 = jnp.sum(e * e, axis=1, keepdims=True).reshape(1, -1)
    iota_f = lax.broadcasted_iota(jnp.int32, (1, K), 1).astype(jnp.float32)
    ones_r = jnp.ones((1, Rb), jnp.float32)
    idx_cols = []
    tile_counts = jnp.zeros((1, K), jnp.float32)
    tile_commit = jnp.zeros((), jnp.float32)
    for c in range(NUM_CODEBOOKS):
        zc = z_ref[:, c * Dc:(c + 1) * Dc]
        z2 = jnp.sum(zc * zc, axis=1, keepdims=True)
        s_neg = lax.dot_general(zc * -2.0, e, (((1,), (1,)), ((), ())),
                                preferred_element_type=jnp.float32)
        d = (z2 + s_neg) + e2
        md = jnp.min(d, axis=1, keepdims=True)
        idxf = jnp.min(jnp.where(d == md, iota_f, 2048.0), axis=1,
                       keepdims=True)
        idx_cols.append(idxf.astype(jnp.int32))
        onehot = (iota_f == idxf).astype(jnp.float32)
        tile_counts += lax.dot_general(ones_r, onehot,
                                       (((1,), (0,)), ((), ())),
                                       preferred_element_type=jnp.float32)
        tile_commit += jnp.sum(md)
    idx_ref[...] = jnp.concatenate(idx_cols, axis=1)

    @pl.when(i == 0)
    def _init():
        counts_scr[...] = tile_counts
        commit_scr[0] = tile_commit

    @pl.when(i > 0)
    def _acc():
        counts_scr[...] += tile_counts
        commit_scr[0] += tile_commit

    @pl.when(i == pl.num_programs(0) - 1)
    def _fin():
        total_rows = Rb * NUM_CODEBOOKS * pl.num_programs(0)
        p = counts_scr[...] / total_rows
        ent = -jnp.sum(p * jnp.log(p + EPS), axis=1, keepdims=True)
        ppl_ref[...] = jnp.exp(ent)
        commit_ref[...] = jnp.full((1, 1), commit_scr[0], jnp.float32)


ROWS_BT_PER_TILE = 512


def _vq_distance_argmin(z_bt, embed):
    nb, ld = z_bt.shape
    k, dc = embed.shape
    g = nb // ROWS_BT_PER_TILE
    idx, commit, ppl = pl.pallas_call(
        _vq_tc_body,
        grid=(g,),
        in_specs=[
            pl.BlockSpec((ROWS_BT_PER_TILE, ld), lambda i: (i, 0)),
            pl.BlockSpec((k, dc), lambda i: (0, 0)),
        ],
        out_specs=[
            pl.BlockSpec((ROWS_BT_PER_TILE, NUM_CODEBOOKS), lambda i: (i, 0)),
            pl.BlockSpec((1, 1), lambda i: (0, 0)),
            pl.BlockSpec((1, 1), lambda i: (0, 0)),
        ],
        out_shape=[
            jax.ShapeDtypeStruct((nb, NUM_CODEBOOKS), jnp.int32),
            jax.ShapeDtypeStruct((1, 1), jnp.float32),
            jax.ShapeDtypeStruct((1, 1), jnp.float32),
        ],
        scratch_shapes=[
            pltpu.VMEM((1, k), jnp.float32),
            pltpu.SMEM((1,), jnp.float32),
        ],
        compiler_params=pltpu.CompilerParams(
            dimension_semantics=("arbitrary",)),
    )(z_bt, embed)
    return idx, commit[0, 0], ppl[0, 0]


_SC_CHUNK = 128


def _make_sc_gather(n_rows, dc):
    info = plsc.get_sparse_core_info()
    nw = info.num_cores * info.num_subcores
    b_per_w = n_rows // nw
    n_ch = b_per_w // _SC_CHUNK
    mesh = plsc.VectorSubcoreMesh(core_axis_name="c", subcore_axis_name="s")

    @functools.partial(
        pl.kernel, mesh=mesh,
        out_type=jax.ShapeDtypeStruct((n_rows, dc), jnp.float32),
        scratch_types=[
            pltpu.VMEM((n_ch, _SC_CHUNK), jnp.int32),
            pltpu.VMEM((_SC_CHUNK, dc), jnp.float32),
            pltpu.VMEM((_SC_CHUNK, dc), jnp.float32),
            pltpu.SemaphoreType.DMA,
            pltpu.SemaphoreType.DMA,
        ],
    )
    def _gather(idx_hbm, table_hbm, out_hbm, idx_v, rows_a, rows_b, sem_a,
                sem_b):
        wid = lax.axis_index("s") * info.num_cores + lax.axis_index("c")
        base = wid * b_per_w
        pltpu.sync_copy(idx_hbm.at[pl.ds(wid * n_ch, n_ch)], idx_v)
        bufs = (rows_a, rows_b)
        sems = (sem_a, sem_b)
        copies = [None] * n_ch
        copies[0] = pltpu.async_copy(table_hbm.at[idx_v.at[0]], bufs[0],
                                     sems[0])
        for c in range(n_ch):
            if c + 1 < n_ch:
                copies[c + 1] = pltpu.async_copy(
                    table_hbm.at[idx_v.at[c + 1]], bufs[(c + 1) % 2],
                    sems[(c + 1) % 2])
            copies[c].wait()
            pltpu.sync_copy(bufs[c % 2],
                            out_hbm.at[pl.ds(base + c * _SC_CHUNK, _SC_CHUNK)])

    return _gather


def kernel(z_bt, embed):
    k, dc = embed.shape
    n = z_bt.shape[0] * NUM_CODEBOOKS

    indices, commit_sum, perplexity = _vq_distance_argmin(z_bt, embed)

    idx2d = indices.reshape(-1, _SC_CHUNK)
    quantized_flat = _make_sc_gather(n, dc)(idx2d, embed)
    quantized_st = quantized_flat.reshape(z_bt.shape)

    commitment_loss = commit_sum / z_bt.size
    codebook_loss = jnp.zeros((), dtype=z_bt.dtype)
    loss = COMMITMENT_COST * commitment_loss
    return (quantized_st, indices, loss, codebook_loss, commitment_loss,
            perplexity)


# speedup vs baseline: 1.5166x; 1.2958x over previous
import functools

import jax
import jax.numpy as jnp
from jax import lax
from jax.experimental import pallas as pl
from jax.experimental.pallas import tpu as pltpu
from jax.experimental.pallas import tpu_sc as plsc

NUM_EMBEDDINGS = 1024
CODE_DIM = 256
NUM_CODEBOOKS = 4
COMMITMENT_COST = 0.25
EPS = 1e-10

def _vq_tc_body(z_ref, e_ref, idx_ref, commit_ref, ppl_ref, counts_scr,
                commit_scr):
    i = pl.program_id(0)
    e = e_ref[...]
    K, Dc = e.shape
    Rb = z_ref.shape[0]
    e2 = jnp.sum(e * e, axis=1, keepdims=True).reshape(1, -1)
    iota_f = lax.broadcasted_iota(jnp.int32, (1, K), 1).astype(jnp.float32)
    ones_r = jnp.ones((1, Rb), jnp.float32)
    idx_cols = []
    tile_counts = jnp.zeros((1, K), jnp.float32)
    tile_commit = jnp.zeros((), jnp.float32)
    for c in range(NUM_CODEBOOKS):
        zc = z_ref[:, c * Dc:(c + 1) * Dc]
        z2 = jnp.sum(zc * zc, axis=1, keepdims=True)
        s_neg = lax.dot_general(zc * -2.0, e, (((1,), (1,)), ((), ())),
                                preferred_element_type=jnp.float32)
        d = (z2 + s_neg) + e2
        md = jnp.min(d, axis=1, keepdims=True)
        idxf = jnp.min(jnp.where(d == md, iota_f, 2048.0), axis=1,
                       keepdims=True)
        idx_cols.append(idxf.astype(jnp.int32))
        onehot = (iota_f == idxf).astype(jnp.float32)
        tile_counts += lax.dot_general(ones_r, onehot,
                                       (((1,), (0,)), ((), ())),
                                       preferred_element_type=jnp.float32)
        tile_commit += jnp.sum(md)
    idx_ref[...] = jnp.concatenate(idx_cols, axis=1)

    @pl.when(i == 0)
    def _init():
        counts_scr[...] = tile_counts
        commit_scr[0] = tile_commit

    @pl.when(i > 0)
    def _acc():
        counts_scr[...] += tile_counts
        commit_scr[0] += tile_commit

    @pl.when(i == pl.num_programs(0) - 1)
    def _fin():
        total_rows = Rb * NUM_CODEBOOKS * pl.num_programs(0)
        p = counts_scr[...] / total_rows
        ent = -jnp.sum(p * jnp.log(p + EPS), axis=1, keepdims=True)
        ppl_ref[...] = jnp.exp(ent)
        commit_ref[...] = jnp.full((1, 1), commit_scr[0], jnp.float32)


ROWS_BT_PER_TILE = 512


def _vq_distance_argmin(z_bt, embed):
    nb, ld = z_bt.shape
    k, dc = embed.shape
    g = nb // ROWS_BT_PER_TILE
    idx, commit, ppl = pl.pallas_call(
        _vq_tc_body,
        grid=(g,),
        in_specs=[
            pl.BlockSpec((ROWS_BT_PER_TILE, ld), lambda i: (i, 0)),
            pl.BlockSpec((k, dc), lambda i: (0, 0)),
        ],
        out_specs=[
            pl.BlockSpec((ROWS_BT_PER_TILE, NUM_CODEBOOKS), lambda i: (i, 0)),
            pl.BlockSpec((1, 1), lambda i: (0, 0)),
            pl.BlockSpec((1, 1), lambda i: (0, 0)),
        ],
        out_shape=[
            jax.ShapeDtypeStruct((nb, NUM_CODEBOOKS), jnp.int32),
            jax.ShapeDtypeStruct((1, 1), jnp.float32),
            jax.ShapeDtypeStruct((1, 1), jnp.float32),
        ],
        scratch_shapes=[
            pltpu.VMEM((1, k), jnp.float32),
            pltpu.SMEM((1,), jnp.float32),
        ],
        compiler_params=pltpu.CompilerParams(
            dimension_semantics=("arbitrary",)),
    )(z_bt, embed)
    return idx, commit[0, 0], ppl[0, 0]


_SC_ROWS = 32


def _make_sc_gather(nb, ld, dc):
    info = plsc.get_sparse_core_info()
    nw = info.num_cores * info.num_subcores
    rows_per_w = nb // nw
    n_ch = rows_per_w // _SC_ROWS
    ncb = ld // dc
    mesh = plsc.VectorSubcoreMesh(core_axis_name="c", subcore_axis_name="s")

    @functools.partial(
        pl.kernel, mesh=mesh,
        out_type=jax.ShapeDtypeStruct((nb, ld), jnp.float32),
        scratch_types=[
            pltpu.VMEM((ncb, 2, 128), jnp.int32),
            pltpu.VMEM((_SC_ROWS, ld), jnp.float32),
            pltpu.VMEM((_SC_ROWS, ld), jnp.float32),
            pltpu.SemaphoreType.DMA,
            pltpu.SemaphoreType.DMA,
        ],
    )
    def _gather(idx_hbm, table_hbm, out_hbm, idx_v, buf_a, buf_b, sem_a,
                sem_b):
        wid = lax.axis_index("s") * info.num_cores + lax.axis_index("c")
        base = wid * rows_per_w
        pltpu.sync_copy(idx_hbm.at[:, pl.ds(2 * wid, 2), :], idx_v)

        def chunk_gathers(m, buf, sem):
            j = m // 4
            off = (m % 4) * _SC_ROWS
            return [
                pltpu.async_copy(
                    table_hbm.at[idx_v.at[c, j, pl.ds(off, _SC_ROWS)]],
                    buf.at[:, pl.ds(c * dc, dc)], sem)
                for c in range(ncb)
            ]

        def body(t, carry):
            m0 = 2 * t
            cps_a = chunk_gathers(m0, buf_a, sem_a)
            cps_b = chunk_gathers(m0 + 1, buf_b, sem_b)
            for cp in cps_a:
                cp.wait()
            pltpu.sync_copy(buf_a,
                            out_hbm.at[pl.ds(base + m0 * _SC_ROWS, _SC_ROWS)])
            for cp in cps_b:
                cp.wait()
            pltpu.sync_copy(
                buf_b, out_hbm.at[pl.ds(base + (m0 + 1) * _SC_ROWS,
                                        _SC_ROWS)])
            return carry

        lax.fori_loop(0, n_ch // 2, body, 0)

    return _gather


def kernel(z_bt, embed):
    k, dc = embed.shape
    nb, ld = z_bt.shape

    indices, commit_sum, perplexity = _vq_distance_argmin(z_bt, embed)

    idx_t = jnp.transpose(indices).reshape(ld // dc, nb // 128, 128)
    quantized_st = _make_sc_gather(nb, ld, dc)(idx_t, embed)

    commitment_loss = commit_sum / z_bt.size
    codebook_loss = jnp.zeros((), dtype=z_bt.dtype)
    loss = COMMITMENT_COST * commitment_loss
    return (quantized_st, indices, loss, codebook_loss, commitment_loss,
            perplexity)
